# baseline (speedup 1.0000x reference)
import functools

import jax
import jax.numpy as jnp
from jax import lax
from jax.experimental import pallas as pl
from jax.experimental.pallas import tpu as pltpu
from jax.experimental.pallas import tpu_sc as plsc

D_MODEL = 128


def _build_sc_gather(V, E, B, S):
    info = plsc.get_sparse_core_info()
    NC, NS = info.num_cores, info.num_subcores
    NW = NC * NS
    n_groups = NW // S
    assert B % (8 * n_groups) == 0
    b_per_g = B // n_groups
    mesh = plsc.VectorSubcoreMesh(core_axis_name="c", subcore_axis_name="s")

    @functools.partial(
        pl.kernel,
        out_type=jax.ShapeDtypeStruct((B, S * E), jnp.float32),
        mesh=mesh,
        compiler_params=pltpu.CompilerParams(
            use_tc_tiling_on_sc=False, needs_layout_passes=False
        ),
        scratch_types=[
            pltpu.VMEM((b_per_g,), jnp.int32),
            pltpu.VMEM((b_per_g, E), jnp.float32),
            pltpu.SemaphoreType.DMA,
        ],
    )
    def gather_kernel(
        table_hbm, i0_hbm, i1_hbm, i2_hbm, i3_hbm, out_hbm, idx_v, rows_v, sem
    ):
        wid = lax.axis_index("s") * NC + lax.axis_index("c")
        s = wid % S
        base = (wid // S) * b_per_g
        idx_refs = [i0_hbm, i1_hbm, i2_hbm, i3_hbm]
        for si in range(S):
            @pl.when(s == si)
            def _():
                pltpu.sync_copy(idx_refs[si].at[pl.ds(base, b_per_g)], idx_v)
        pltpu.async_copy(table_hbm.at[idx_v], rows_v, sem).wait()
        pltpu.sync_copy(
            rows_v, out_hbm.at[pl.ds(base, b_per_g), pl.ds(s * E, E)]
        )

    return gather_kernel


def _build_sc_detile(V, E):
    info = plsc.get_sparse_core_info()
    NW = info.num_cores * info.num_subcores
    NSLAB = (V + 511) // 512
    NSLAB_FULL = NSLAB - 1
    TAIL_CHUNKS = (V - NSLAB_FULL * 512) // 16
    TAIL_TILES = 2
    NJ = (NSLAB + NW - 1) // NW
    mesh = plsc.VectorSubcoreMesh(core_axis_name="c", subcore_axis_name="s")

    @functools.partial(
        pl.kernel,
        out_type=jax.ShapeDtypeStruct((V // 4, 4 * E), jnp.float32),
        mesh=mesh,
        compiler_params=pltpu.CompilerParams(
            use_tc_tiling_on_sc=True,
            needs_layout_passes=False,
            disable_bounds_checks=True,
        ),
        scratch_types=[
            pltpu.VMEM((128, 128), jnp.float32),
            pltpu.VMEM((128, 128), jnp.float32),
            pltpu.VMEM((128, 128), jnp.float32),
            pltpu.VMEM((128, 128), jnp.float32),
            pltpu.SemaphoreType.DMA,
            pltpu.SemaphoreType.DMA,
        ],
    )
    def detile_kernel(tt_hbm, out_hbm, st_a, st_b, ov_a, ov_b, sem_i, sem_o):
        wid = lax.axis_index("s") * info.num_cores + lax.axis_index("c")
        lanes = lax.iota(jnp.int32, 16)
        dpbase = lax.shift_right_logical(lanes, 2)
        cbase = lax.shift_left(lanes & 3, 5)
        stages = [st_a, st_b]
        outs = [ov_a, ov_b]

        def fire_stage(s, stage_v, n_tiles):
            for F in range(4):
                for tp in range(n_tiles):
                    pltpu.async_copy(
                        tt_hbm.at[
                            pl.ds(8 * F, 8), pl.ds((s * 4 + tp) * 128, 128)
                        ],
                        stage_v.at[pl.ds(F * 32 + 8 * tp, 8), :],
                        sem_i,
                    )

        def wait_stage(s, stage_v, n_tiles):
            for F in range(4):
                for tp in range(n_tiles):
                    pltpu.make_async_copy(
                        tt_hbm.at[
                            pl.ds(8 * F, 8), pl.ds((s * 4 + tp) * 128, 128)
                        ],
                        stage_v.at[pl.ds(F * 32 + 8 * tp, 8), :],
                        sem_i,
                    ).wait()

        def emit(stage_v, out_v, n_chunks):
            def chunk(c, _):
                rbase = lax.shift_right_logical(c, 3) * 8
                colb = lax.shift_left(c & 7, 4)
                cload = lanes + colb
                rvec = jnp.full((16,), rbase, jnp.int32)
                dp_vec = dpbase + 4 * c
                for f in range(32):
                    vals = plsc.load_gather(
                        stage_v, [rvec + ((f // 8) * 32 + (f % 8)), cload]
                    )
                    plsc.store_scatter(out_v, [dp_vec, cbase + f], vals)
                return _

            lax.fori_loop(0, n_chunks, chunk, None)

        def fire_out(s, out_v, n_rows):
            pltpu.async_copy(
                out_v.at[pl.ds(0, n_rows)],
                out_hbm.at[pl.ds(s * 128, n_rows)],
                sem_o,
            )

        def wait_out(s, out_v, n_rows):
            pltpu.make_async_copy(
                out_v.at[pl.ds(0, n_rows)],
                out_hbm.at[pl.ds(s * 128, n_rows)],
                sem_o,
            ).wait()

        def slab_params(j):
            s = wid + NW * j
            return s, s < NSLAB_FULL, s == NSLAB_FULL

        def do_stage(j):
            s, full, tail = slab_params(j)
            pl.when(full)(lambda: fire_stage(s, stages[j % 2], 4))
            pl.when(tail)(lambda: fire_stage(s, stages[j % 2], TAIL_TILES))

        def do_emit_and_out(j):
            s, full, tail = slab_params(j)

            def _full():
                wait_stage(s, stages[j % 2], 4)
                emit(stages[j % 2], outs[j % 2], 32)
                fire_out(s, outs[j % 2], 128)

            def _tail():
                wait_stage(s, stages[j % 2], TAIL_TILES)
                emit(stages[j % 2], outs[j % 2], TAIL_CHUNKS)
                fire_out(s, outs[j % 2], 40)

            pl.when(full)(_full)
            pl.when(tail)(_tail)

        def drain_out(j):
            s, full, tail = slab_params(j)
            pl.when(full)(lambda: wait_out(s, outs[j % 2], 128))
            pl.when(tail)(lambda: wait_out(s, outs[j % 2], 40))

        do_stage(0)
        for j in range(NJ):
            if j + 1 < NJ:
                do_stage(j + 1)
            if j >= 2:
                drain_out(j - 2)
            do_emit_and_out(j)
        for j in range(max(NJ - 2, 0), NJ):
            drain_out(j)

    return detile_kernel


def _mm_body(e_ref, w_ref, b_ref, o_ref):
    o_ref[...] = lax.dot_general(
        e_ref[...], w_ref[...],
        dimension_numbers=(((1,), (1,)), ((), ())),
        preferred_element_type=jnp.float32,
    ) + b_ref[...]


def _tc_project(e, W, b2d, block_m):
    B = e.shape[0]
    return pl.pallas_call(
        _mm_body,
        out_shape=jax.ShapeDtypeStruct((B, D_MODEL), jnp.float32),
        grid=(B // block_m,),
        in_specs=[
            pl.BlockSpec((block_m, D_MODEL), lambda i: (i, 0)),
            pl.BlockSpec((D_MODEL, D_MODEL), lambda i: (0, 0)),
            pl.BlockSpec((1, D_MODEL), lambda i: (0, 0)),
        ],
        out_specs=pl.BlockSpec((block_m, D_MODEL), lambda i: (i, 0)),
    )(e, W, b2d)


def kernel(ctrl_tokens, embed_table, W, b):
    B, S = ctrl_tokens.shape
    V, E = embed_table.shape
    idx = ctrl_tokens.astype(jnp.int32)
    cols = [idx[:, s] for s in range(S)]
    table_lin = _build_sc_detile(V, E)(embed_table.T).reshape(V, E)
    e = _build_sc_gather(V, E, B, S)(table_lin, *cols)
    out = _tc_project(e, W, b.reshape(1, D_MODEL), 2048)
    return out[..., None]


# speedup vs baseline: 1.4249x; 1.4249x over previous
import functools

import jax
import jax.numpy as jnp
from jax import lax
from jax.experimental import pallas as pl
from jax.experimental.pallas import tpu as pltpu
from jax.experimental.pallas import tpu_sc as plsc

D_MODEL = 128


def _build_sc_gather(V, E, B, S):
    info = plsc.get_sparse_core_info()
    NC, NS = info.num_cores, info.num_subcores
    NW = NC * NS
    n_groups = NW // S
    assert B % (8 * n_groups) == 0
    b_per_g = B // n_groups
    mesh = plsc.VectorSubcoreMesh(core_axis_name="c", subcore_axis_name="s")

    @functools.partial(
        pl.kernel,
        out_type=jax.ShapeDtypeStruct((B, S * E), jnp.float32),
        mesh=mesh,
        compiler_params=pltpu.CompilerParams(
            use_tc_tiling_on_sc=False, needs_layout_passes=False
        ),
        scratch_types=[
            pltpu.VMEM((b_per_g,), jnp.int32),
            pltpu.VMEM((b_per_g, E), jnp.float32),
            pltpu.SemaphoreType.DMA,
        ],
    )
    def gather_kernel(
        table_hbm, i0_hbm, i1_hbm, i2_hbm, i3_hbm, out_hbm, idx_v, rows_v, sem
    ):
        wid = lax.axis_index("s") * NC + lax.axis_index("c")
        s = wid % S
        base = (wid // S) * b_per_g
        idx_refs = [i0_hbm, i1_hbm, i2_hbm, i3_hbm]
        for si in range(S):
            @pl.when(s == si)
            def _():
                pltpu.sync_copy(idx_refs[si].at[pl.ds(base, b_per_g)], idx_v)
        pltpu.async_copy(table_hbm.at[idx_v], rows_v, sem).wait()
        pltpu.sync_copy(
            rows_v, out_hbm.at[pl.ds(base, b_per_g), pl.ds(s * E, E)]
        )

    return gather_kernel


def _mm_body(e_ref, w_ref, b_ref, o_ref):
    o_ref[...] = lax.dot_general(
        e_ref[...], w_ref[...],
        dimension_numbers=(((1,), (1,)), ((), ())),
        preferred_element_type=jnp.float32,
    ) + b_ref[...]


def _tc_project(e, W, b2d, block_m):
    B = e.shape[0]
    return pl.pallas_call(
        _mm_body,
        out_shape=jax.ShapeDtypeStruct((B, D_MODEL), jnp.float32),
        grid=(B // block_m,),
        in_specs=[
            pl.BlockSpec((block_m, D_MODEL), lambda i: (i, 0)),
            pl.BlockSpec((D_MODEL, D_MODEL), lambda i: (0, 0)),
            pl.BlockSpec((1, D_MODEL), lambda i: (0, 0)),
        ],
        out_specs=pl.BlockSpec((block_m, D_MODEL), lambda i: (i, 0)),
    )(e, W, b2d)


def kernel(ctrl_tokens, embed_table, W, b):
    B, S = ctrl_tokens.shape
    V, E = embed_table.shape
    idx = ctrl_tokens.astype(jnp.int32)
    cols = [idx[:, s] for s in range(S)]
    e = _build_sc_gather(V, E, B, S)(embed_table, *cols)
    out = _tc_project(e, W, b.reshape(1, D_MODEL), 2048)
    return out[..., None]
